# baseline (device time: 86544 ns/iter reference)
import jax
import jax.numpy as jnp
from jax import lax
from jax.experimental import pallas as pl
from jax.experimental.pallas import tpu as pltpu

N_DEV = 32
P = 8
Z = 4
ROWS = 1024
PCH = ROWS // P
ZCH = PCH // Z
NSTEP = (P - 1) + (Z - 1) + (Z - 1) + (P - 1)
DH = 128
SCALE = 0.08838834764831843
BF = jnp.bfloat16
F32 = jnp.float32


def _ar_body(
    p_ref, out_ref,
    prs_s, prs_r,
    zrs_s, zrs_r,
    zag_s, zag_r,
    pag_s, pag_r,
    ss, rs,
):
    my = lax.axis_index("i")
    z = my // P
    q = lax.rem(my, P)

    plane_peers = [z * P + lax.rem(q + j, P) for j in range(1, P)]
    z_peers = [lax.rem(z + j, Z) * P + q for j in range(1, Z)]

    barrier = pltpu.get_barrier_semaphore()
    for nbr in plane_peers + z_peers:
        pl.semaphore_signal(
            barrier, inc=1, device_id=(nbr,), device_id_type=pl.DeviceIdType.MESH
        )
    pl.semaphore_wait(barrier, len(plane_peers) + len(z_peers))

    pending = []

    def start(src, dst, ssem, rsem, target):
        r = pltpu.make_async_remote_copy(
            src_ref=src,
            dst_ref=dst,
            send_sem=ssem,
            recv_sem=rsem,
            device_id=(target,),
            device_id_type=pl.DeviceIdType.MESH,
        )
        r.start()
        pending.append(r)
        return r

    prs_s[...] = p_ref[...].astype(BF)
    flows = []
    for j in range(1, P):
        e = lax.rem(q + j, P)
        flows.append(
            start(prs_s.at[pl.ds(e * PCH, PCH), :], prs_r.at[j - 1],
                  ss.at[j - 1], rs.at[j - 1], z * P + e)
        )
    own = q * PCH
    for d in flows:
        d.wait_recv()
    acc = prs_r[0].astype(F32)
    for j in range(1, P - 1):
        acc = acc + prs_r[j].astype(F32)
    out_ref[pl.ds(own, PCH), :] = p_ref[pl.ds(own, PCH), :] + acc

    k0 = P - 1
    flows = []
    for j in range(1, Z):
        ez = lax.rem(z + j, Z)
        zrs_s[j - 1, :, :] = out_ref[pl.ds(own + ez * ZCH, ZCH), :].astype(BF)
        flows.append(
            start(zrs_s.at[j - 1], zrs_r.at[j - 1], ss.at[k0 + j - 1],
                  rs.at[k0 + j - 1], ez * P + q)
        )
    zown = own + z * ZCH
    for d in flows:
        d.wait_recv()
    zacc = zrs_r[0].astype(F32)
    for j in range(1, Z - 1):
        zacc = zacc + zrs_r[j].astype(F32)
    out_ref[pl.ds(zown, ZCH), :] = out_ref[pl.ds(zown, ZCH), :] + zacc

    k0 = (P - 1) + (Z - 1)
    zag_s[...] = out_ref[pl.ds(zown, ZCH), :].astype(BF)
    flows = []
    for j in range(1, Z):
        flows.append(
            start(zag_s, zag_r.at[j - 1], ss.at[k0 + j - 1], rs.at[k0 + j - 1],
                  lax.rem(z + j, Z) * P + q)
        )
    for j, d in enumerate(flows):
        d.wait_recv()
        src_z = lax.rem(z - j - 1 + Z, Z)
        out_ref[pl.ds(own + src_z * ZCH, ZCH), :] = zag_r[j].astype(F32)

    k0 = (P - 1) + 2 * (Z - 1)
    pag_s[...] = out_ref[pl.ds(own, PCH), :].astype(BF)
    flows = []
    for j in range(1, P):
        flows.append(
            start(pag_s, pag_r.at[j - 1], ss.at[k0 + j - 1], rs.at[k0 + j - 1],
                  z * P + lax.rem(q + j, P))
        )
    for j, d in enumerate(flows):
        d.wait_recv()
        src_q = lax.rem(q - j - 1 + P, P)
        out_ref[pl.ds(src_q * PCH, PCH), :] = pag_r[j].astype(F32)

    for r in pending:
        r.wait_send()


def _hier_allreduce(partial):
    return pl.pallas_call(
        _ar_body,
        out_shape=jax.ShapeDtypeStruct((ROWS, ROWS), F32),
        in_specs=[pl.BlockSpec(memory_space=pltpu.VMEM)],
        out_specs=pl.BlockSpec(memory_space=pltpu.VMEM),
        scratch_shapes=[
            pltpu.VMEM((ROWS, ROWS), BF),
            pltpu.VMEM((P - 1, PCH, ROWS), BF),
            pltpu.VMEM((Z - 1, ZCH, ROWS), BF),
            pltpu.VMEM((Z - 1, ZCH, ROWS), BF),
            pltpu.VMEM((ZCH, ROWS), BF),
            pltpu.VMEM((Z - 1, ZCH, ROWS), BF),
            pltpu.VMEM((PCH, ROWS), BF),
            pltpu.VMEM((P - 1, PCH, ROWS), BF),
            pltpu.SemaphoreType.DMA((NSTEP,)),
            pltpu.SemaphoreType.DMA((NSTEP,)),
        ],
        compiler_params=pltpu.CompilerParams(collective_id=0),
    )(partial)


def kernel(x, Wq, K_ext, V_ext, Wo):
    my = lax.axis_index("i")
    hl = Wq.shape[1] // DH

    x2 = x[0].astype(BF)
    Q = jnp.dot(x2, Wq.astype(BF), preferred_element_type=F32)
    Q = Q.reshape(ROWS, hl, DH).astype(BF)
    K = lax.dynamic_slice_in_dim(K_ext[0], my * hl, hl, axis=1).astype(BF)
    V = lax.dynamic_slice_in_dim(V_ext[0], my * hl, hl, axis=1).astype(BF)

    def group(t):
        t = t.reshape(4, 4, 64, hl, DH)
        return t.transpose(1, 0, 2, 3, 4).reshape(4, 256, hl, DH)

    Qg, Kg, Vg = group(Q), group(K), group(V)
    scores = (
        jnp.einsum("gihd,gjhd->ghij", Qg, Kg, preferred_element_type=F32) * SCALE
    )
    w = jax.nn.softmax(scores, axis=-1).astype(BF)
    ctx = jnp.einsum("ghij,gjhd->gihd", w, Vg, preferred_element_type=F32)
    ctx = (
        ctx.reshape(4, 4, 64, hl, DH)
        .transpose(1, 0, 2, 3, 4)
        .reshape(ROWS, hl * DH)
        .astype(BF)
    )
    partial = jnp.dot(ctx, Wo.astype(BF), preferred_element_type=F32)

    out = _hier_allreduce(partial)
    return out.reshape(1, ROWS, ROWS)


# device time: 85763 ns/iter; 1.0091x vs baseline; 1.0091x over previous
import jax
import jax.numpy as jnp
from jax import lax
from jax.experimental import pallas as pl
from jax.experimental.pallas import tpu as pltpu

N_DEV = 32
P = 8
Z = 4
ROWS = 1024
PCH = ROWS // P
ZCH = PCH // Z
NSTEP = (P - 1) + (Z - 1) + (Z - 1) + (P - 1)
DH = 128
SCALE = 0.08838834764831843
BF = jnp.bfloat16
F32 = jnp.float32


def _ar_body(
    ctx_ref, wo_ref, out_ref,
    prs_s, prs_r,
    zrs_s, zrs_r,
    zag_s, zag_r,
    pag_s, pag_r,
    ss, rs,
):
    my = lax.axis_index("i")
    z = my // P
    q = lax.rem(my, P)

    plane_peers = [z * P + lax.rem(q + j, P) for j in range(1, P)]
    z_peers = [lax.rem(z + j, Z) * P + q for j in range(1, Z)]

    barrier = pltpu.get_barrier_semaphore()
    for nbr in plane_peers + z_peers:
        pl.semaphore_signal(
            barrier, inc=1, device_id=(nbr,), device_id_type=pl.DeviceIdType.MESH
        )
    pl.semaphore_wait(barrier, len(plane_peers) + len(z_peers))

    pending = []

    def start(src, dst, ssem, rsem, target):
        r = pltpu.make_async_remote_copy(
            src_ref=src,
            dst_ref=dst,
            send_sem=ssem,
            recv_sem=rsem,
            device_id=(target,),
            device_id_type=pl.DeviceIdType.MESH,
        )
        r.start()
        pending.append(r)
        return r

    flows = []
    for j in range(1, P):
        e = lax.rem(q + j, P)
        pc = jnp.dot(
            ctx_ref[pl.ds(e * PCH, PCH), :], wo_ref[...],
            preferred_element_type=F32,
        )
        prs_s[pl.ds(e * PCH, PCH), :] = pc.astype(BF)
        flows.append(
            start(prs_s.at[pl.ds(e * PCH, PCH), :], prs_r.at[j - 1],
                  ss.at[j - 1], rs.at[j - 1], z * P + e)
        )
    own = q * PCH
    own_part = jnp.dot(
        ctx_ref[pl.ds(own, PCH), :], wo_ref[...], preferred_element_type=F32
    )
    for d in flows:
        d.wait_recv()
    acc = prs_r[0].astype(F32)
    for j in range(1, P - 1):
        acc = acc + prs_r[j].astype(F32)
    out_ref[pl.ds(own, PCH), :] = own_part + acc

    k0 = P - 1
    flows = []
    for j in range(1, Z):
        ez = lax.rem(z + j, Z)
        zrs_s[j - 1, :, :] = out_ref[pl.ds(own + ez * ZCH, ZCH), :].astype(BF)
        flows.append(
            start(zrs_s.at[j - 1], zrs_r.at[j - 1], ss.at[k0 + j - 1],
                  rs.at[k0 + j - 1], ez * P + q)
        )
    zown = own + z * ZCH
    for d in flows:
        d.wait_recv()
    zacc = zrs_r[0].astype(F32)
    for j in range(1, Z - 1):
        zacc = zacc + zrs_r[j].astype(F32)
    out_ref[pl.ds(zown, ZCH), :] = out_ref[pl.ds(zown, ZCH), :] + zacc

    k0 = (P - 1) + (Z - 1)
    zag_s[...] = out_ref[pl.ds(zown, ZCH), :].astype(BF)
    flows = []
    for j in range(1, Z):
        flows.append(
            start(zag_s, zag_r.at[j - 1], ss.at[k0 + j - 1], rs.at[k0 + j - 1],
                  lax.rem(z + j, Z) * P + q)
        )
    for j, d in enumerate(flows):
        d.wait_recv()
        src_z = lax.rem(z - j - 1 + Z, Z)
        out_ref[pl.ds(own + src_z * ZCH, ZCH), :] = zag_r[j].astype(F32)

    k0 = (P - 1) + 2 * (Z - 1)
    pag_s[...] = out_ref[pl.ds(own, PCH), :].astype(BF)
    flows = []
    for j in range(1, P):
        flows.append(
            start(pag_s, pag_r.at[j - 1], ss.at[k0 + j - 1], rs.at[k0 + j - 1],
                  z * P + lax.rem(q + j, P))
        )
    for j, d in enumerate(flows):
        d.wait_recv()
        src_q = lax.rem(q - j - 1 + P, P)
        out_ref[pl.ds(src_q * PCH, PCH), :] = pag_r[j].astype(F32)

    for r in pending:
        r.wait_send()


def _proj_allreduce(ctx, wo):
    return pl.pallas_call(
        _ar_body,
        out_shape=jax.ShapeDtypeStruct((ROWS, ROWS), F32),
        in_specs=[pl.BlockSpec(memory_space=pltpu.VMEM)] * 2,
        out_specs=pl.BlockSpec(memory_space=pltpu.VMEM),
        scratch_shapes=[
            pltpu.VMEM((ROWS, ROWS), BF),
            pltpu.VMEM((P - 1, PCH, ROWS), BF),
            pltpu.VMEM((Z - 1, ZCH, ROWS), BF),
            pltpu.VMEM((Z - 1, ZCH, ROWS), BF),
            pltpu.VMEM((ZCH, ROWS), BF),
            pltpu.VMEM((Z - 1, ZCH, ROWS), BF),
            pltpu.VMEM((PCH, ROWS), BF),
            pltpu.VMEM((P - 1, PCH, ROWS), BF),
            pltpu.SemaphoreType.DMA((NSTEP,)),
            pltpu.SemaphoreType.DMA((NSTEP,)),
        ],
        compiler_params=pltpu.CompilerParams(collective_id=0),
    )(ctx, wo)


def kernel(x, Wq, K_ext, V_ext, Wo):
    my = lax.axis_index("i")
    hl = Wq.shape[1] // DH

    x2 = x[0].astype(BF)
    Q = jnp.dot(x2, Wq.astype(BF), preferred_element_type=F32)
    Q = Q.reshape(ROWS, hl, DH).astype(BF)
    K = lax.dynamic_slice_in_dim(K_ext[0], my * hl, hl, axis=1).astype(BF)
    V = lax.dynamic_slice_in_dim(V_ext[0], my * hl, hl, axis=1).astype(BF)

    def group(t):
        t = t.reshape(4, 4, 64, hl, DH)
        return t.transpose(1, 0, 2, 3, 4).reshape(4, 256, hl, DH)

    Qg, Kg, Vg = group(Q), group(K), group(V)
    scores = (
        jnp.einsum("gihd,gjhd->ghij", Qg, Kg, preferred_element_type=F32) * SCALE
    )
    w = jax.nn.softmax(scores, axis=-1).astype(BF)
    ctx = jnp.einsum("ghij,gjhd->gihd", w, Vg, preferred_element_type=F32)
    ctx = (
        ctx.reshape(4, 4, 64, hl, DH)
        .transpose(1, 0, 2, 3, 4)
        .reshape(ROWS, hl * DH)
        .astype(BF)
    )

    out = _proj_allreduce(ctx, Wo.astype(BF))
    return out.reshape(1, ROWS, ROWS)
